# baseline (speedup 1.0000x reference)
import functools

import jax
import jax.numpy as jnp
from jax import lax
from jax.experimental import pallas as pl
from jax.experimental.pallas import tpu as pltpu
from jax.experimental.pallas import tpu_sc as plsc

VOCAB = 8192
D = 8192
NC, NS = 2, 16
NW = NC * NS
BTOT = 16 * 256
BPW = BTOT // NW
K = 2
NCH = BPW // K
NB = 4


def _gather_body(idx_hbm, tbl_hbm, out_hbm, idx_v,
                 buf0, buf1, buf2, buf3,
                 sg0, sg1, sg2, sg3, so0, so1, so2, so3):
    wid = lax.axis_index("s") * NC + lax.axis_index("c")
    base = wid * BPW
    pltpu.sync_copy(idx_hbm.at[wid], idx_v)
    bufs = (buf0, buf1, buf2, buf3)
    sgs = (sg0, sg1, sg2, sg3)
    sos = (so0, so1, so2, so3)

    def start_g(c, b):
        pltpu.async_copy(tbl_hbm.at[idx_v.at[c]], bufs[b], sgs[b])

    def wait_g(c, b):
        pltpu.make_async_copy(tbl_hbm.at[idx_v.at[c]], bufs[b], sgs[b]).wait()

    def start_o(c, b):
        pltpu.async_copy(bufs[b], out_hbm.at[pl.ds(base + c * K, K)], sos[b])

    def wait_o(c, b):
        pltpu.make_async_copy(
            bufs[b], out_hbm.at[pl.ds(base + c * K, K)], sos[b]).wait()

    for b in range(NB):
        start_g(b, b)

    for c in range(2):
        wait_g(c, c)
        start_o(c, c)

    def quad_body(i, carry):
        c0 = 4 * i + 2
        for j in range(4):
            c = c0 + j
            b = (2 + j) % NB
            bo = (b + 2) % NB
            wait_o(c - 2, bo)
            start_g(c + 2, bo)
            wait_g(c, b)
            start_o(c, b)
        return carry

    lax.fori_loop(0, (NCH - 4) // 4, quad_body, 0)

    for c in range(NCH - 2, NCH):
        wait_o(c - 2, (c + 2) % NB)
        wait_g(c, c % NB)
        start_o(c, c % NB)
    for c in range(NCH - 2, NCH):
        wait_o(c, c % NB)


_sc_gather = functools.partial(
    pl.kernel,
    mesh=plsc.VectorSubcoreMesh(core_axis_name="c", subcore_axis_name="s"),
    out_type=jax.ShapeDtypeStruct((BTOT, D), jnp.float32),
    scratch_types=[
        pltpu.VMEM((NCH, K), jnp.int32),
        pltpu.VMEM((K, D), jnp.float32),
        pltpu.VMEM((K, D), jnp.float32),
        pltpu.VMEM((K, D), jnp.float32),
        pltpu.VMEM((K, D), jnp.float32),
        pltpu.SemaphoreType.DMA,
        pltpu.SemaphoreType.DMA,
        pltpu.SemaphoreType.DMA,
        pltpu.SemaphoreType.DMA,
        pltpu.SemaphoreType.DMA,
        pltpu.SemaphoreType.DMA,
        pltpu.SemaphoreType.DMA,
        pltpu.SemaphoreType.DMA,
    ],
)(_gather_body)


def kernel(x, embed_weight):
    B, L = x.shape
    idx = x.reshape(NW, NCH, K).astype(jnp.int32)
    out = _sc_gather(idx, embed_weight)
    return out.reshape(B, L, D)


# speedup vs baseline: 1.0103x; 1.0103x over previous
import functools

import jax
import jax.numpy as jnp
from jax import lax
from jax.experimental import pallas as pl
from jax.experimental.pallas import tpu as pltpu
from jax.experimental.pallas import tpu_sc as plsc

VOCAB = 8192
D = 8192
NC, NS = 2, 16
NW = NC * NS
BTOT = 16 * 256
BPW = BTOT // NW
K = 2
NCH = BPW // K
HALF = NCH // 2


def _gather_body(idx_hbm, tbl_hbm, out_hbm, idx_v, d0, d1, e0, e1, sp,
                 sgd0, sgd1, sod0, sod1, sgs0, sgs1, sos0, sos1):
    cid = lax.axis_index("c")
    sid = lax.axis_index("s")
    wid = sid * NC + cid
    base = wid * BPW
    pltpu.sync_copy(idx_hbm.at[wid], idx_v)
    dbufs = (d0, d1)
    ebufs = (e0, e1)
    sgd = (sgd0, sgd1)
    sod = (sod0, sod1)
    sgs = (sgs0, sgs1)
    sos = (sos0, sos1)

    def start_g(c, b):
        pltpu.async_copy(tbl_hbm.at[idx_v.at[c]], dbufs[b], sgd[b])

    def wait_g(c, b):
        pltpu.make_async_copy(tbl_hbm.at[idx_v.at[c]], dbufs[b], sgd[b]).wait()

    def start_o(c, b):
        pltpu.async_copy(dbufs[b], out_hbm.at[pl.ds(base + c * K, K)], sod[b])

    def wait_o(c, b):
        pltpu.make_async_copy(
            dbufs[b], out_hbm.at[pl.ds(base + c * K, K)], sod[b]).wait()

    def start_spg(c, b):
        pltpu.async_copy(tbl_hbm.at[idx_v.at[c]], ebufs[b], sgs[b])

    def wait_spg(c, b):
        pltpu.make_async_copy(
            tbl_hbm.at[idx_v.at[c]], ebufs[b], sgs[b]).wait()

    def sync_x(b):
        pltpu.sync_copy(ebufs[b], sp.at[sid, b])

    def start_spo(c, b):
        pltpu.async_copy(
            sp.at[sid, b], out_hbm.at[pl.ds(base + c * K, K)], sos[b])

    def wait_spo(c, b):
        pltpu.make_async_copy(
            sp.at[sid, b], out_hbm.at[pl.ds(base + c * K, K)], sos[b]).wait()

    start_g(0, 0)
    start_g(1, 1)
    start_spg(HALF + 0, 0)
    start_spg(HALF + 1, 1)
    wait_g(0, 0)
    start_o(0, 0)
    wait_spg(HALF + 0, 0)
    sync_x(0)
    start_spo(HALF + 0, 0)
    start_spg(HALF + 2, 0)
    wait_o(0, 0)
    start_g(2, 0)
    wait_g(1, 1)
    start_o(1, 1)
    wait_spg(HALF + 1, 1)
    sync_x(1)
    start_spo(HALF + 1, 1)
    start_spg(HALF + 3, 1)

    def step(i, b):
        s = HALF + i
        wait_o(i - 1, 1 - b)
        start_g(i + 1, 1 - b)
        wait_g(i, b)
        start_o(i, b)
        wait_spo(s - 2, b)
        wait_spg(s, b)
        sync_x(b)
        start_spo(s, b)
        start_spg(s + 2, b)

    def pair_body(j, carry):
        i = 2 * j + 2
        step(i, 0)
        step(i + 1, 1)
        return carry

    lax.fori_loop(0, (HALF - 4) // 2, pair_body, 0)

    i = HALF - 2
    wait_o(i - 1, 1)
    start_g(i + 1, 1)
    wait_g(i, 0)
    start_o(i, 0)
    wait_spo(HALF + i - 2, 0)
    wait_spg(HALF + i, 0)
    sync_x(0)
    start_spo(HALF + i, 0)
    i = HALF - 1
    wait_o(i - 1, 0)
    wait_g(i, 1)
    start_o(i, 1)
    wait_spo(HALF + i - 2, 1)
    wait_spg(HALF + i, 1)
    sync_x(1)
    start_spo(HALF + i, 1)
    wait_o(HALF - 1, 1)
    wait_spo(NCH - 2, 0)
    wait_spo(NCH - 1, 1)


_sc_gather = functools.partial(
    pl.kernel,
    mesh=plsc.VectorSubcoreMesh(core_axis_name="c", subcore_axis_name="s"),
    out_type=jax.ShapeDtypeStruct((BTOT, D), jnp.float32),
    scratch_types=[
        pltpu.VMEM((NCH, K), jnp.int32),
        pltpu.VMEM((K, D), jnp.float32),
        pltpu.VMEM((K, D), jnp.float32),
        pltpu.VMEM((K, D), jnp.float32),
        pltpu.VMEM((K, D), jnp.float32),
        pltpu.VMEM_SHARED((NS, 2, K, D), jnp.float32),
        pltpu.SemaphoreType.DMA,
        pltpu.SemaphoreType.DMA,
        pltpu.SemaphoreType.DMA,
        pltpu.SemaphoreType.DMA,
        pltpu.SemaphoreType.DMA,
        pltpu.SemaphoreType.DMA,
        pltpu.SemaphoreType.DMA,
        pltpu.SemaphoreType.DMA,
    ],
)(_gather_body)


def kernel(x, embed_weight):
    B, L = x.shape
    idx = x.reshape(NW, NCH, K).astype(jnp.int32)
    out = _sc_gather(idx, embed_weight)
    return out.reshape(B, L, D)


# speedup vs baseline: 1.0138x; 1.0035x over previous
import functools

import jax
import jax.numpy as jnp
from jax import lax
from jax.experimental import pallas as pl
from jax.experimental.pallas import tpu as pltpu
from jax.experimental.pallas import tpu_sc as plsc

VOCAB = 8192
D = 8192
NC, NS = 2, 16
NW = NC * NS
BTOT = 16 * 256
BPW = BTOT // NW
K = 2
NCH = BPW // K
HALF = NCH // 2


def _gather_body(idx_hbm, tbl_hbm, out_hbm, idx_v, d0, d1, e0, e1, sp,
                 sgd0, sgd1, sod0, sod1, sgs0, sgs1, sos0, sos1):
    cid = lax.axis_index("c")
    sid = lax.axis_index("s")
    wid = sid * NC + cid
    base = wid * BPW
    pltpu.sync_copy(idx_hbm.at[wid], idx_v)
    dbufs = (d0, d1)
    ebufs = (e0, e1)
    sgd = (sgd0, sgd1)
    sod = (sod0, sod1)
    sgs = (sgs0, sgs1)
    sos = (sos0, sos1)

    def start_g(c, b):
        pltpu.async_copy(tbl_hbm.at[idx_v.at[c]], dbufs[b], sgd[b])

    def wait_g(c, b):
        pltpu.make_async_copy(tbl_hbm.at[idx_v.at[c]], dbufs[b], sgd[b]).wait()

    def start_o(c, b):
        pltpu.async_copy(dbufs[b], out_hbm.at[pl.ds(base + c * K, K)], sod[b])

    def wait_o(c, b):
        pltpu.make_async_copy(
            dbufs[b], out_hbm.at[pl.ds(base + c * K, K)], sod[b]).wait()

    def start_spg(c, b):
        pltpu.async_copy(tbl_hbm.at[idx_v.at[c]], ebufs[b], sgs[b])

    def wait_spg(c, b):
        pltpu.make_async_copy(
            tbl_hbm.at[idx_v.at[c]], ebufs[b], sgs[b]).wait()

    def sync_x(b):
        pltpu.sync_copy(ebufs[b], sp.at[sid, b])

    def start_spo(c, b):
        pltpu.async_copy(
            sp.at[sid, b], out_hbm.at[pl.ds(base + c * K, K)], sos[b])

    def wait_spo(c, b):
        pltpu.make_async_copy(
            sp.at[sid, b], out_hbm.at[pl.ds(base + c * K, K)], sos[b]).wait()

    start_g(0, 0)
    start_g(1, 1)
    start_spg(HALF + 0, 0)
    start_spg(HALF + 1, 1)
    wait_g(0, 0)
    start_o(0, 0)
    wait_spg(HALF + 0, 0)
    sync_x(0)
    start_spo(HALF + 0, 0)
    start_spg(HALF + 2, 0)
    wait_o(0, 0)
    start_g(2, 0)
    wait_g(1, 1)
    start_o(1, 1)
    wait_spg(HALF + 1, 1)
    sync_x(1)
    start_spo(HALF + 1, 1)
    start_spg(HALF + 3, 1)

    def step(i, b):
        s = HALF + i
        wait_o(i - 1, 1 - b)
        start_g(i + 1, 1 - b)
        wait_g(i, b)
        start_o(i, b)
        wait_spo(s - 2, b)
        wait_spg(s, b)
        sync_x(b)
        start_spo(s, b)
        start_spg(s + 2, b)

    def pair_body(j, carry):
        i = 2 * j + 2
        step(i, 0)
        step(i + 1, 1)
        return carry

    lax.fori_loop(0, (HALF - 4) // 2, pair_body, 0)

    i = HALF - 2
    wait_o(i - 1, 1)
    start_g(i + 1, 1)
    wait_g(i, 0)
    start_o(i, 0)
    wait_spo(HALF + i - 2, 0)
    wait_spg(HALF + i, 0)
    sync_x(0)
    start_spo(HALF + i, 0)
    i = HALF - 1
    wait_o(i - 1, 0)
    wait_g(i, 1)
    start_o(i, 1)
    wait_spo(HALF + i - 2, 1)
    wait_spg(HALF + i, 1)
    sync_x(1)
    start_spo(HALF + i, 1)
    wait_o(HALF - 1, 1)
    wait_spo(NCH - 2, 0)
    wait_spo(NCH - 1, 1)


_sc_gather = functools.partial(
    pl.kernel,
    mesh=plsc.VectorSubcoreMesh(core_axis_name="c", subcore_axis_name="s"),
    out_type=jax.ShapeDtypeStruct((BTOT, D), jnp.float32),
    scratch_types=[
        pltpu.VMEM((NCH, K), jnp.int32),
        pltpu.VMEM((K, D), jnp.float32),
        pltpu.VMEM((K, D), jnp.float32),
        pltpu.VMEM((K, D), jnp.float32),
        pltpu.VMEM((K, D), jnp.float32),
        pltpu.VMEM_SHARED((NS, 2, K, D), jnp.float32),
        pltpu.SemaphoreType.DMA,
        pltpu.SemaphoreType.DMA,
        pltpu.SemaphoreType.DMA,
        pltpu.SemaphoreType.DMA,
        pltpu.SemaphoreType.DMA,
        pltpu.SemaphoreType.DMA,
        pltpu.SemaphoreType.DMA,
        pltpu.SemaphoreType.DMA,
    ],
)(_gather_body)


def kernel(x, embed_weight):
    B, L = x.shape
    idx = x.reshape(NW, NCH, K)
    if idx.dtype != jnp.int32:
        idx = idx.astype(jnp.int32)
    out = _sc_gather(idx, embed_weight)
    return out.reshape(B, L, D)


# speedup vs baseline: 1.0176x; 1.0037x over previous
import functools

import jax
import jax.numpy as jnp
from jax import lax
from jax.experimental import pallas as pl
from jax.experimental.pallas import tpu as pltpu
from jax.experimental.pallas import tpu_sc as plsc

VOCAB = 8192
D = 8192
NC, NS = 2, 16
NW = NC * NS
BTOT = 16 * 256
BPW = BTOT // NW
K = 2
NCH = BPW // K
HALF = NCH // 2


def _gather_body(idx_hbm, tbl_hbm, out_hbm, idx_v, d0, d1, e0, e1, sp,
                 sgd0, sgd1, sod0, sod1, sgs0, sgs1, sos0, sos1):
    cid = lax.axis_index("c")
    sid = lax.axis_index("s")
    wid = sid * NC + cid
    base = wid * BPW
    pltpu.sync_copy(
        idx_hbm.at[pl.ds(wid // 2, 1),
                   pl.ds(pl.multiple_of((wid % 2) * BPW, BPW), BPW)],
        idx_v)
    dbufs = (d0, d1)
    ebufs = (e0, e1)
    sgd = (sgd0, sgd1)
    sod = (sod0, sod1)
    sgs = (sgs0, sgs1)
    sos = (sos0, sos1)

    def idx_at(c):
        return idx_v.at[0, pl.ds(c * K, K)]

    def start_g(c, b):
        pltpu.async_copy(tbl_hbm.at[idx_at(c)], dbufs[b], sgd[b])

    def wait_g(c, b):
        pltpu.make_async_copy(tbl_hbm.at[idx_at(c)], dbufs[b], sgd[b]).wait()

    def start_o(c, b):
        pltpu.async_copy(dbufs[b], out_hbm.at[pl.ds(base + c * K, K)], sod[b])

    def wait_o(c, b):
        pltpu.make_async_copy(
            dbufs[b], out_hbm.at[pl.ds(base + c * K, K)], sod[b]).wait()

    def start_spg(c, b):
        pltpu.async_copy(tbl_hbm.at[idx_at(c)], ebufs[b], sgs[b])

    def wait_spg(c, b):
        pltpu.make_async_copy(
            tbl_hbm.at[idx_at(c)], ebufs[b], sgs[b]).wait()

    def sync_x(b):
        pltpu.sync_copy(ebufs[b], sp.at[sid, b])

    def start_spo(c, b):
        pltpu.async_copy(
            sp.at[sid, b], out_hbm.at[pl.ds(base + c * K, K)], sos[b])

    def wait_spo(c, b):
        pltpu.make_async_copy(
            sp.at[sid, b], out_hbm.at[pl.ds(base + c * K, K)], sos[b]).wait()

    start_g(0, 0)
    start_g(1, 1)
    start_spg(HALF + 0, 0)
    start_spg(HALF + 1, 1)
    wait_g(0, 0)
    start_o(0, 0)
    wait_spg(HALF + 0, 0)
    sync_x(0)
    start_spo(HALF + 0, 0)
    start_spg(HALF + 2, 0)
    wait_o(0, 0)
    start_g(2, 0)
    wait_g(1, 1)
    start_o(1, 1)
    wait_spg(HALF + 1, 1)
    sync_x(1)
    start_spo(HALF + 1, 1)
    start_spg(HALF + 3, 1)

    def step(i, b):
        s = HALF + i
        wait_o(i - 1, 1 - b)
        start_g(i + 1, 1 - b)
        wait_g(i, b)
        start_o(i, b)
        wait_spo(s - 2, b)
        wait_spg(s, b)
        sync_x(b)
        start_spo(s, b)
        start_spg(s + 2, b)

    def pair_body(j, carry):
        i = 2 * j + 2
        step(i, 0)
        step(i + 1, 1)
        return carry

    lax.fori_loop(0, (HALF - 4) // 2, pair_body, 0)

    i = HALF - 2
    wait_o(i - 1, 1)
    start_g(i + 1, 1)
    wait_g(i, 0)
    start_o(i, 0)
    wait_spo(HALF + i - 2, 0)
    wait_spg(HALF + i, 0)
    sync_x(0)
    start_spo(HALF + i, 0)
    i = HALF - 1
    wait_o(i - 1, 0)
    wait_g(i, 1)
    start_o(i, 1)
    wait_spo(HALF + i - 2, 1)
    wait_spg(HALF + i, 1)
    sync_x(1)
    start_spo(HALF + i, 1)
    wait_o(HALF - 1, 1)
    wait_spo(NCH - 2, 0)
    wait_spo(NCH - 1, 1)


_sc_gather = functools.partial(
    pl.kernel,
    mesh=plsc.VectorSubcoreMesh(core_axis_name="c", subcore_axis_name="s"),
    out_type=jax.ShapeDtypeStruct((BTOT, D), jnp.float32),
    scratch_types=[
        pltpu.VMEM((1, BPW), jnp.int32),
        pltpu.VMEM((K, D), jnp.float32),
        pltpu.VMEM((K, D), jnp.float32),
        pltpu.VMEM((K, D), jnp.float32),
        pltpu.VMEM((K, D), jnp.float32),
        pltpu.VMEM_SHARED((NS, 2, K, D), jnp.float32),
        pltpu.SemaphoreType.DMA,
        pltpu.SemaphoreType.DMA,
        pltpu.SemaphoreType.DMA,
        pltpu.SemaphoreType.DMA,
        pltpu.SemaphoreType.DMA,
        pltpu.SemaphoreType.DMA,
        pltpu.SemaphoreType.DMA,
        pltpu.SemaphoreType.DMA,
    ],
)(_gather_body)


def kernel(x, embed_weight):
    B, L = x.shape
    if x.dtype != jnp.int32:
        x = x.astype(jnp.int32)
    out = _sc_gather(x, embed_weight)
    return out.reshape(B, L, D)
